# baseline (speedup 1.0000x reference)
import functools

import jax
import jax.numpy as jnp
from jax import lax
from jax.experimental import pallas as pl
from jax.experimental.pallas import tpu as pltpu
from jax.experimental.pallas import tpu_sc as plsc

_DIM = 32
_NB = 16384
_S = 50
_QMAX = 31.0
_NW = 32
_ROWS = _NB * _S
_PW = _ROWS // _NW
_CHUNK = 800
_NCHUNK = _PW // _CHUNK
_MAGIC = 1.5 * 2.0 ** 23


def _round_ne(x):
    return (x + _MAGIC) - _MAGIC


_GATHER_DNUMS = lax.GatherDimensionNumbers(
    offset_dims=(), collapsed_slice_dims=(0,), start_index_map=(0,)
)


def _shuffle(v, idx):
    return lax.gather(
        v,
        idx[:, None],
        _GATHER_DNUMS,
        slice_sizes=(1,),
        mode=lax.GatherScatterMode.PROMISE_IN_BOUNDS,
    )


def _lane_max_all(m):
    lanes = lax.iota(jnp.int32, 16)
    for d in (8, 4, 2, 1):
        m = jnp.maximum(m, _shuffle(m, lanes ^ d))
    return m


def _quantize_inplace(buf):
    @plsc.parallel_loop(0, _CHUNK, unroll=4)
    def body(r):
        v0 = buf[r, pl.ds(0, 16)]
        v1 = buf[r, pl.ds(16, 16)]
        m = _lane_max_all(jnp.maximum(jnp.abs(v0), jnp.abs(v1)))
        scale = m / _QMAX
        scale = jnp.where(scale == 0.0, 1.0, scale)
        inv = 1.0 / scale
        buf[r, pl.ds(0, 16)] = _round_ne(v0 * inv) * scale
        buf[r, pl.ds(16, 16)] = _round_ne(v1 * inv) * scale


def _make_kernel():
    mesh = plsc.VectorSubcoreMesh(core_axis_name="c", subcore_axis_name="s")

    @functools.partial(
        pl.kernel,
        out_type=jax.ShapeDtypeStruct((_ROWS, _DIM), jnp.float32),
        mesh=mesh,
        scratch_types=[
            pltpu.VMEM((_CHUNK,), jnp.int32),
            pltpu.VMEM((_CHUNK,), jnp.int32),
            pltpu.VMEM((_CHUNK, _DIM), jnp.float32),
            pltpu.VMEM((_CHUNK, _DIM), jnp.float32),
            pltpu.SemaphoreType.DMA,
            pltpu.SemaphoreType.DMA,
        ],
        compiler_params=pltpu.CompilerParams(use_tc_tiling_on_sc=False),
    )
    def k(table_hbm, idx_hbm, out_hbm, idxs0, idxs1, buf0, buf1,
          gsem0, gsem1):
        wid = lax.axis_index("s") * 2 + lax.axis_index("c")
        r0 = wid * _PW
        idxs = (idxs0, idxs1)
        buf = (buf0, buf1)
        gsem = (gsem0, gsem1)

        def fire_gather(c, p):
            start = r0 + c * _CHUNK
            pltpu.sync_copy(idx_hbm.at[pl.ds(start, _CHUNK)], idxs[p])
            pltpu.async_copy(table_hbm.at[idxs[p]], buf[p], gsem[p])

        def step(c, p):
            pltpu.make_async_copy(
                table_hbm.at[idxs[p]], buf[p], gsem[p]
            ).wait()
            _quantize_inplace(buf[p])
            start = r0 + c * _CHUNK
            pltpu.sync_copy(buf[p], out_hbm.at[pl.ds(start, _CHUNK)])

        fire_gather(0, 0)
        fire_gather(1, 1)

        def body(i, carry):
            c0 = 2 * i
            step(c0, 0)

            @pl.when(i < _NCHUNK // 2 - 1)
            def _():
                fire_gather(c0 + 2, 0)

            step(c0 + 1, 1)

            @pl.when(i < _NCHUNK // 2 - 1)
            def _():
                fire_gather(c0 + 3, 1)

            return carry

        lax.fori_loop(0, _NCHUNK // 2, body, jnp.int32(0))

    return k


_kernel_call = _make_kernel()


@jax.jit
def kernel(input, weight):
    idx = input.reshape(-1).astype(jnp.int32)
    out = _kernel_call(weight, idx)
    return out.reshape(_NB, _S, _DIM)


# speedup vs baseline: 1.0048x; 1.0048x over previous
import functools

import jax
import jax.numpy as jnp
from jax import lax
from jax.experimental import pallas as pl
from jax.experimental.pallas import tpu as pltpu
from jax.experimental.pallas import tpu_sc as plsc

_DIM = 32
_NB = 16384
_S = 50
_QMAX = 31.0
_NW = 32
_ROWS = _NB * _S
_PW = _ROWS // _NW
_CHUNK = 1600
_NCHUNK = _PW // _CHUNK
_MAGIC = 1.5 * 2.0 ** 23


def _round_ne(x):
    return (x + _MAGIC) - _MAGIC


_GATHER_DNUMS = lax.GatherDimensionNumbers(
    offset_dims=(), collapsed_slice_dims=(0,), start_index_map=(0,)
)


def _shuffle(v, idx):
    return lax.gather(
        v,
        idx[:, None],
        _GATHER_DNUMS,
        slice_sizes=(1,),
        mode=lax.GatherScatterMode.PROMISE_IN_BOUNDS,
    )


def _lane_max_all(m):
    lanes = lax.iota(jnp.int32, 16)
    for d in (8, 4, 2, 1):
        m = jnp.maximum(m, _shuffle(m, lanes ^ d))
    return m


def _quantize_inplace(buf):
    @plsc.parallel_loop(0, _CHUNK, unroll=4)
    def body(r):
        v0 = buf[r, pl.ds(0, 16)]
        v1 = buf[r, pl.ds(16, 16)]
        m = _lane_max_all(jnp.maximum(jnp.abs(v0), jnp.abs(v1)))
        scale = m / _QMAX
        scale = jnp.where(scale == 0.0, 1.0, scale)
        inv = 1.0 / scale
        buf[r, pl.ds(0, 16)] = _round_ne(v0 * inv) * scale
        buf[r, pl.ds(16, 16)] = _round_ne(v1 * inv) * scale


def _make_kernel():
    mesh = plsc.VectorSubcoreMesh(core_axis_name="c", subcore_axis_name="s")

    @functools.partial(
        pl.kernel,
        out_type=jax.ShapeDtypeStruct((_ROWS, _DIM), jnp.float32),
        mesh=mesh,
        scratch_types=[
            pltpu.VMEM((_CHUNK,), jnp.int32),
            pltpu.VMEM((_CHUNK,), jnp.int32),
            pltpu.VMEM((_CHUNK, _DIM), jnp.float32),
            pltpu.VMEM((_CHUNK, _DIM), jnp.float32),
            pltpu.SemaphoreType.DMA,
            pltpu.SemaphoreType.DMA,
        ],
        compiler_params=pltpu.CompilerParams(use_tc_tiling_on_sc=False),
    )
    def k(table_hbm, idx_hbm, out_hbm, idxs0, idxs1, buf0, buf1,
          gsem0, gsem1):
        wid = lax.axis_index("s") * 2 + lax.axis_index("c")
        r0 = wid * _PW
        idxs = (idxs0, idxs1)
        buf = (buf0, buf1)
        gsem = (gsem0, gsem1)

        def fire_gather(c, p):
            start = r0 + c * _CHUNK
            pltpu.sync_copy(idx_hbm.at[pl.ds(start, _CHUNK)], idxs[p])
            pltpu.async_copy(table_hbm.at[idxs[p]], buf[p], gsem[p])

        def step(c, p):
            pltpu.make_async_copy(
                table_hbm.at[idxs[p]], buf[p], gsem[p]
            ).wait()
            _quantize_inplace(buf[p])
            start = r0 + c * _CHUNK
            pltpu.sync_copy(buf[p], out_hbm.at[pl.ds(start, _CHUNK)])

        fire_gather(0, 0)
        fire_gather(1, 1)

        def body(i, carry):
            c0 = 2 * i
            step(c0, 0)

            @pl.when(i < _NCHUNK // 2 - 1)
            def _():
                fire_gather(c0 + 2, 0)

            step(c0 + 1, 1)

            @pl.when(i < _NCHUNK // 2 - 1)
            def _():
                fire_gather(c0 + 3, 1)

            return carry

        lax.fori_loop(0, _NCHUNK // 2, body, jnp.int32(0))

    return k


_kernel_call = _make_kernel()


@jax.jit
def kernel(input, weight):
    idx = input.reshape(-1).astype(jnp.int32)
    out = _kernel_call(weight, idx)
    return out.reshape(_NB, _S, _DIM)
